# Initial kernel scaffold; baseline (speedup 1.0000x reference)
import jax
import jax.numpy as jnp
from jax.experimental import pallas as pl


def kernel(hm_feature, reg_feature, rad_feature, stride, infer_topk):
    raise NotImplementedError("write your pallas kernel here")


# speedup vs baseline: 5.7818x; 5.7818x over previous
import functools

import jax
import jax.numpy as jnp
from jax.experimental import pallas as pl
from jax.experimental.pallas import tpu as pltpu

_D = _H = _W = 96
_LANES = 128
_TOPK = 256
_CONF = 0.3
_NT = _H // 8
_BIG = 2**30


def _body(hm_ref, reg_ref, rad_ref, stride_ref, out_ref, ms_ref):
    neg_inf = jnp.float32(-jnp.inf)
    s = jax.nn.sigmoid(hm_ref[...])

    cw = jnp.full((_D, _H, 1), neg_inf, jnp.float32)
    mw = jnp.maximum(
        s,
        jnp.maximum(jnp.concatenate([s[:, :, 1:], cw], axis=2),
                    jnp.concatenate([cw, s[:, :, :-1]], axis=2)))
    ch = jnp.full((_D, 1, _W), neg_inf, jnp.float32)
    mh = jnp.maximum(
        mw,
        jnp.maximum(jnp.concatenate([mw[:, 1:, :], ch], axis=1),
                    jnp.concatenate([ch, mw[:, :-1, :]], axis=1)))
    cd = jnp.full((1, _H, _W), neg_inf, jnp.float32)
    md = jnp.maximum(
        mh,
        jnp.maximum(jnp.concatenate([mh[1:], cd], axis=0),
                    jnp.concatenate([cd, mh[:-1]], axis=0)))

    keep = (md == s) & (s > _CONF)
    masked = jnp.where(keep, s, neg_inf)
    masked = jnp.concatenate(
        [masked, jnp.full((_D, _H, _LANES - _W), neg_inf, jnp.float32)], axis=2)
    m4 = masked.reshape(_D, _NT, 8, _LANES)
    ms_ref[...] = m4
    rowmax = jnp.max(jnp.max(m4, axis=3), axis=2)

    iota_zt = (jax.lax.broadcasted_iota(jnp.int32, (_D, _NT), 0) * _NT
               + jax.lax.broadcasted_iota(jnp.int32, (_D, _NT), 1))
    iota_v = (jax.lax.broadcasted_iota(jnp.int32, (8, _LANES), 0) * _LANES
              + jax.lax.broadcasted_iota(jnp.int32, (8, _LANES), 1))
    lane_iota = jax.lax.broadcasted_iota(jnp.int32, (1, _W), 1)
    lane8 = jax.lax.broadcasted_iota(jnp.int32, (1, 8), 1)
    stride_s = stride_ref[0, 0]

    def step(i, rowmax):
        m = jnp.max(rowmax)
        rt = jnp.min(jnp.where(rowmax == m, iota_zt, _BIG))
        zr = rt // _NT
        tr = rt % _NT
        v = ms_ref[pl.ds(zr, 1), pl.ds(tr, 1)].reshape(8, _LANES)
        p = jnp.min(jnp.where(v == m, iota_v, _BIG))
        sy = p // _LANES
        sx = p % _LANES
        y = tr * 8 + sy
        v2 = jnp.where(iota_v == p, neg_inf, v)
        ms_ref[pl.ds(zr, 1), pl.ds(tr, 1)] = v2.reshape(1, 1, 8, _LANES)
        rowmax = jnp.where(iota_zt == rt, jnp.max(v2), rowmax)

        sel = (lane_iota == sx).astype(jnp.float32)
        r0 = jnp.sum(reg_ref[pl.ds(0, 1), pl.ds(zr, 1), pl.ds(y, 1), :]
                     .reshape(1, _W) * sel)
        r1 = jnp.sum(reg_ref[pl.ds(1, 1), pl.ds(zr, 1), pl.ds(y, 1), :]
                     .reshape(1, _W) * sel)
        r2 = jnp.sum(reg_ref[pl.ds(2, 1), pl.ds(zr, 1), pl.ds(y, 1), :]
                     .reshape(1, _W) * sel)
        rd = jnp.sum(rad_ref[pl.ds(zr, 1), pl.ds(y, 1), :]
                     .reshape(1, _W) * sel)
        b1 = (r0 + zr.astype(jnp.float32)) * stride_s
        b2 = (r1 + y.astype(jnp.float32)) * stride_s
        b3 = (r2 + sx.astype(jnp.float32)) * stride_s
        rr = jnp.exp(rd)
        row = jnp.where(lane8 == 0, m,
              jnp.where(lane8 == 1, b1,
              jnp.where(lane8 == 2, b2,
              jnp.where(lane8 == 3, b3,
              jnp.where(lane8 == 4, rr, 0.0)))))
        row = jnp.where(m > _CONF, row, jnp.zeros_like(row))
        out_ref[pl.ds(i, 1), :] = row
        return rowmax

    jax.lax.fori_loop(0, _TOPK, step, rowmax)


@functools.partial(jax.jit, static_argnames=())
def kernel(hm_feature, reg_feature, rad_feature, stride, infer_topk):
    del infer_topk
    hm = hm_feature[0, 0]
    reg = reg_feature[0]
    rad = rad_feature[0, 0]
    stride_arr = jnp.asarray(stride, jnp.float32).reshape(1, 1)
    out = pl.pallas_call(
        _body,
        out_shape=jax.ShapeDtypeStruct((_TOPK, 8), jnp.float32),
        scratch_shapes=[pltpu.VMEM((_D, _NT, 8, _LANES), jnp.float32)],
    )(hm, reg, rad, stride_arr)
    return out[:, :5]
